# baseline (speedup 1.0000x reference)
import jax
import jax.numpy as jnp
from jax.experimental import pallas as pl
from jax.experimental.pallas import tpu as pltpu

_BF = jnp.bfloat16


def _dot(a, b):
    return jax.lax.dot_general(a, b, (((1,), (0,)), ((), ())),
                               preferred_element_type=jnp.float32)


def _dot_tn(a, b):
    return jax.lax.dot_general(a, b, (((0,), (0,)), ((), ())),
                               preferred_element_type=jnp.float32)


def _dot_nt(a, b):
    return jax.lax.dot_general(a, b, (((1,), (1,)), ((), ())),
                               preferred_element_type=jnp.float32)


def _body(x_ref, adj_s_ref, adj_t_ref, tw1_ref, tw2_ref, tw3_ref,
          sw1_ref, sw2_ref, sw3_ref, d1w_ref, d1b_ref, d2w_ref, d2b_ref,
          d3w_ref, d3b_ref, fcw_ref, fcb_ref, out_ref):
    xb = x_ref[...].astype(_BF)
    adj_tb = adj_t_ref[...].astype(_BF)
    adj_sb = adj_s_ref[...].astype(_BF)

    d1w = d1w_ref[...].astype(_BF).astype(jnp.float32)
    d2w = d2w_ref[...].astype(_BF).astype(jnp.float32)
    d3w = d3w_ref[...].astype(_BF).astype(jnp.float32)
    m = _dot(_dot(d1w, d2w), d3w)
    b_eff = _dot(_dot(d1b_ref[...], d2w) + d2b_ref[...],
                 d3w) + d3b_ref[...]

    t1 = jax.lax.dot_general(tw1_ref[...].astype(_BF), xb,
                             (((0,), (1,)), ((), ())),
                             preferred_element_type=jnp.float32)
    h = jnp.maximum(_dot_nt(t1.astype(_BF), adj_tb), 0.0)
    h = _dot_tn(tw2_ref[...].astype(_BF), h.astype(_BF))
    h = jnp.maximum(_dot_nt(h.astype(_BF), adj_tb), 0.0)
    xw3 = _dot_tn(h.astype(_BF), tw3_ref[...].astype(_BF))
    i = pl.program_id(0)
    adj_tb_half = adj_t_ref[pl.ds(i * 512, 512), :].astype(_BF)
    x_t = jnp.maximum(_dot(adj_tb_half, xw3.astype(_BF)), 0.0)

    g = _dot_tn(sw1_ref[...].astype(_BF), xb)
    g = jnp.maximum(_dot_nt(g.astype(_BF), adj_sb), 0.0)
    g = _dot_tn(sw2_ref[...].astype(_BF), g.astype(_BF))
    g = jnp.maximum(_dot_nt(g.astype(_BF), adj_sb), 0.0)
    sw3_half = sw3_ref[:, pl.ds(i * 512, 512)]
    xw3s = _dot_tn(g.astype(_BF), sw3_half.astype(_BF))
    x_st = jnp.maximum(
        jax.lax.dot_general(xw3s.astype(_BF), adj_sb,
                            (((0,), (1,)), ((), ())),
                            preferred_element_type=jnp.float32), 0.0)

    x_stc = x_st.astype(_BF).astype(jnp.float32)
    x_tc = x_t.astype(_BF).astype(jnp.float32)
    d1b = d1b_ref[...]
    d2b = d2b_ref[...]
    o1 = [(x_stc * d1w[0, o] + x_tc * d1w[1, o] + d1b[0, o])
          .astype(_BF).astype(jnp.float32) for o in range(8)]
    o2 = []
    for p in range(4):
        acc = o1[0] * d2w[0, p]
        for o in range(1, 8):
            acc = acc + o1[o] * d2w[o, p]
        o2.append((acc + d2b[0, p]).astype(_BF).astype(jnp.float32))
    fused = o2[0] * d3w[0, 0]
    for p in range(1, 4):
        fused = fused + o2[p] * d3w[p, 0]
    fused = fused + d3b_ref[0, 0]

    out_ref[...] = (_dot_nt(fused.astype(_BF), fcw_ref[...].astype(_BF))
                    + fcb_ref[...])


def kernel(x, x_adj_s, x_adj_t, t_W1, t_W2, t_W3, s_W1, s_W2, s_W3,
           dec1_W, dec1_b, dec2_W, dec2_b, dec3_W, dec3_b, fc_W, fc_b):
    T, Kd = x.shape
    ins = (x, x_adj_s, x_adj_t,
           t_W1[0], t_W2[0], t_W3[0], s_W1[0], s_W2[0], s_W3[0],
           dec1_W, dec1_b.reshape(1, 8), dec2_W, dec2_b.reshape(1, 4),
           dec3_W, dec3_b.reshape(1, 1), fc_W, fc_b.reshape(1, Kd))
    out = pl.pallas_call(
        _body,
        grid=(2,),
        out_shape=jax.ShapeDtypeStruct((T, Kd), jnp.float32),
        in_specs=[pl.BlockSpec(a.shape, lambda i: (0,) * a.ndim)
                  for a in ins],
        out_specs=pl.BlockSpec((T // 2, Kd), lambda i: (i, 0)),
        compiler_params=pltpu.CompilerParams(
            dimension_semantics=("parallel",)),
    )(*ins)
    return out


# speedup vs baseline: 1.0918x; 1.0918x over previous
import jax
import jax.numpy as jnp
from jax.experimental import pallas as pl
from jax.experimental.pallas import tpu as pltpu

_BF = jnp.bfloat16


def _dot(a, b):
    return jax.lax.dot_general(a, b, (((1,), (0,)), ((), ())),
                               preferred_element_type=jnp.float32)


def _dot_tn(a, b):
    return jax.lax.dot_general(a, b, (((0,), (0,)), ((), ())),
                               preferred_element_type=jnp.float32)


def _dot_nt(a, b):
    return jax.lax.dot_general(a, b, (((1,), (1,)), ((), ())),
                               preferred_element_type=jnp.float32)


def _body(x_ref, adj_s_ref, adj_t_ref, tw1_ref, tw2_ref, tw3_ref,
          sw1_ref, sw2_ref, sw3_ref, d1w_ref, d1b_ref, d2w_ref, d2b_ref,
          d3w_ref, d3b_ref, fcw_ref, fcb_ref, out_ref):
    xb = x_ref[...].astype(_BF)
    adj_tb = adj_t_ref[...].astype(_BF)
    adj_sb = adj_s_ref[...].astype(_BF)

    d1w = d1w_ref[...].astype(_BF).astype(jnp.float32)
    d2w = d2w_ref[...].astype(_BF).astype(jnp.float32)
    d3w = d3w_ref[...].astype(_BF).astype(jnp.float32)
    m = _dot(_dot(d1w, d2w), d3w)
    b_eff = _dot(_dot(d1b_ref[...], d2w) + d2b_ref[...],
                 d3w) + d3b_ref[...]

    t1 = jax.lax.dot_general(tw1_ref[...].astype(_BF), xb,
                             (((0,), (1,)), ((), ())),
                             preferred_element_type=jnp.float32)
    h = jnp.maximum(_dot_nt(t1.astype(_BF), adj_tb), 0.0)
    h = _dot_tn(tw2_ref[...].astype(_BF), h.astype(_BF))
    h = jnp.maximum(_dot_nt(h.astype(_BF), adj_tb), 0.0)
    xw3 = _dot_tn(h.astype(_BF), tw3_ref[...].astype(_BF))
    x_t = jnp.maximum(_dot(adj_tb, xw3.astype(_BF)), 0.0)

    g = _dot_tn(sw1_ref[...].astype(_BF), xb)
    g = jnp.maximum(_dot_nt(g.astype(_BF), adj_sb), 0.0)
    g = _dot_tn(sw2_ref[...].astype(_BF), g.astype(_BF))
    g = jnp.maximum(_dot_nt(g.astype(_BF), adj_sb), 0.0)
    xw3s = _dot_tn(g.astype(_BF), sw3_ref[...].astype(_BF))
    x_st = jnp.maximum(
        jax.lax.dot_general(xw3s.astype(_BF), adj_sb,
                            (((0,), (1,)), ((), ())),
                            preferred_element_type=jnp.float32), 0.0)

    x_stc = x_st.astype(_BF).astype(jnp.float32)
    x_tc = x_t.astype(_BF).astype(jnp.float32)
    d1b = d1b_ref[...]
    d2b = d2b_ref[...]
    o1 = [(x_stc * d1w[0, o] + x_tc * d1w[1, o] + d1b[0, o])
          .astype(_BF).astype(jnp.float32) for o in range(8)]
    o2 = []
    for p in range(4):
        acc = o1[0] * d2w[0, p]
        for o in range(1, 8):
            acc = acc + o1[o] * d2w[o, p]
        o2.append((acc + d2b[0, p]).astype(_BF).astype(jnp.float32))
    fused = o2[0] * d3w[0, 0]
    for p in range(1, 4):
        fused = fused + o2[p] * d3w[p, 0]
    fused = fused + d3b_ref[0, 0]

    out_ref[...] = (_dot_nt(fused.astype(_BF), fcw_ref[...].astype(_BF))
                    + fcb_ref[...])


def kernel(x, x_adj_s, x_adj_t, t_W1, t_W2, t_W3, s_W1, s_W2, s_W3,
           dec1_W, dec1_b, dec2_W, dec2_b, dec3_W, dec3_b, fc_W, fc_b):
    T, Kd = x.shape
    vmem = pl.BlockSpec(memory_space=pltpu.VMEM)
    out = pl.pallas_call(
        _body,
        out_shape=jax.ShapeDtypeStruct((T, Kd), jnp.float32),
        in_specs=[vmem] * 17,
        out_specs=vmem,
    )(x, x_adj_s, x_adj_t,
      t_W1[0], t_W2[0], t_W3[0], s_W1[0], s_W2[0], s_W3[0],
      dec1_W, dec1_b.reshape(1, 8), dec2_W, dec2_b.reshape(1, 4),
      dec3_W, dec3_b.reshape(1, 1), fc_W, fc_b.reshape(1, Kd))
    return out
